# speedup vs baseline: 4.0044x; 4.0044x over previous
import functools

import jax
import jax.numpy as jnp
from jax import lax
from jax.experimental import pallas as pl
from jax.experimental.pallas import tpu as pltpu
from jax.experimental.pallas import tpu_sc as plsc

NC = 2
NS = 16
NW = NC * NS
CHUNK = 128


def _matmul_body(f_ref, w_ref, o_ref):
    o_ref[0] = jnp.dot(f_ref[...], w_ref[0], preferred_element_type=jnp.float32)


def _add_relu_body(a_ref, b_ref, o_ref):
    o_ref[...] = jnp.maximum(a_ref[0] + b_ref[0], 0.0)


@functools.partial(jax.jit, static_argnames=("nchunk", "npad", "c", "rpt"))
def _sc_gather_scatter(fi, fo, tflat, zeros, *, nchunk, npad, c, rpt):
    mesh = plsc.VectorSubcoreMesh(
        core_axis_name="c", subcore_axis_name="s", num_cores=NC, num_subcores=NS
    )

    def body(iidx_hbm, oidx_hbm, tfeats_hbm, zeros_hbm, out_hbm,
             iidx_v, oidx_v, rows_v, accum_sh, sem):
        cid = lax.axis_index("c")
        sid = lax.axis_index("s")
        wid = cid * NS + sid
        pltpu.sync_copy(iidx_hbm.at[wid], iidx_v)
        pltpu.sync_copy(oidx_hbm.at[wid], oidx_v)
        pltpu.sync_copy(zeros_hbm.at[pl.ds(sid * rpt, rpt)],
                        accum_sh.at[pl.ds(sid * rpt, rpt)])
        plsc.subcore_barrier()

        def chunk_body(j, carry):
            pltpu.async_copy(tfeats_hbm.at[iidx_v.at[j]], rows_v, sem).wait()
            pltpu.sync_copy(rows_v, accum_sh.at[oidx_v.at[j]], add=True)
            return carry

        lax.fori_loop(0, nchunk, chunk_body, 0)
        plsc.subcore_barrier()
        pltpu.sync_copy(accum_sh.at[pl.ds(sid * rpt, rpt)],
                        out_hbm.at[cid, pl.ds(sid * rpt, rpt)])

    return pl.kernel(
        body,
        out_type=jax.ShapeDtypeStruct((NC, npad, c), jnp.float32),
        mesh=mesh,
        scratch_types=[
            pltpu.VMEM((nchunk, CHUNK), jnp.int32),
            pltpu.VMEM((nchunk, CHUNK), jnp.int32),
            pltpu.VMEM((CHUNK, c), jnp.float32),
            pltpu.VMEM_SHARED((npad, c), jnp.float32),
            pltpu.SemaphoreType.DMA,
        ],
    )(fi, fo, tflat, zeros)


def kernel(feats, in_map, out_map, W):
    n, c_in = feats.shape
    k, e = in_map.shape
    c_out = W.shape[-1]

    in32 = in_map.astype(jnp.int32)
    out32 = out_map.astype(jnp.int32)

    bn = 2000
    tfeats = pl.pallas_call(
        _matmul_body,
        grid=(k, n // bn),
        in_specs=[
            pl.BlockSpec((bn, c_in), lambda ki, ni: (ni, 0)),
            pl.BlockSpec((1, c_in, c_out), lambda ki, ni: (ki, 0, 0)),
        ],
        out_specs=pl.BlockSpec((1, bn, c_out), lambda ki, ni: (ki, ni, 0)),
        out_shape=jax.ShapeDtypeStruct((k, n, c_out), jnp.float32),
    )(feats, W)
    tflat = tfeats.reshape(k * n, c_out)

    etot = k * e
    nchunk = -(-etot // (NW * CHUNK))
    ep = NW * nchunk * CHUNK
    npad = ((n + NS + 127) // 128) * 128
    rpt = npad // NS

    offs = (jnp.arange(k, dtype=jnp.int32) * n)[:, None]
    fi = (in32 + offs).reshape(-1)
    fo = out32.reshape(-1)
    fi = jnp.concatenate([fi, jnp.zeros((ep - etot,), jnp.int32)])
    fo = jnp.concatenate([fo, jnp.full((ep - etot,), n, jnp.int32)])
    fi = fi.reshape(NW, nchunk, CHUNK)
    fo = fo.reshape(NW, nchunk, CHUNK)

    zeros = jnp.zeros((npad, c_out), jnp.float32)
    partials = _sc_gather_scatter(
        fi, fo, tflat, zeros, nchunk=nchunk, npad=npad, c=c_out, rpt=rpt
    )

    out = pl.pallas_call(
        _add_relu_body,
        grid=(1,),
        in_specs=[
            pl.BlockSpec((1, npad, c_out), lambda i: (0, 0, 0)),
            pl.BlockSpec((1, npad, c_out), lambda i: (1, 0, 0)),
        ],
        out_specs=pl.BlockSpec((npad, c_out), lambda i: (0, 0)),
        out_shape=jax.ShapeDtypeStruct((npad, c_out), jnp.float32),
    )(partials, partials)
    return out[:n]


# speedup vs baseline: 4.3214x; 1.0791x over previous
import functools

import jax
import jax.numpy as jnp
from jax import lax
from jax.experimental import pallas as pl
from jax.experimental.pallas import tpu as pltpu
from jax.experimental.pallas import tpu_sc as plsc

NC = 2
NS = 16
NW = NC * NS
CHUNK = 128
INNER = 2


def _matmul_body(f_ref, w_ref, o_ref):
    o_ref[0] = jnp.dot(f_ref[...], w_ref[0], preferred_element_type=jnp.float32)


def _add_relu_body(a_ref, b_ref, o_ref):
    o_ref[...] = jnp.maximum(a_ref[0] + b_ref[0], 0.0)


@functools.partial(jax.jit, static_argnames=("nouter", "npad", "c", "rpt"))
def _sc_gather_scatter(idx, tflat, zeros, *, nouter, npad, c, rpt):
    mesh = plsc.VectorSubcoreMesh(
        core_axis_name="c", subcore_axis_name="s", num_cores=NC, num_subcores=NS
    )
    inner = INNER

    def body(idx_hbm, tfeats_hbm, zeros_hbm, out_hbm,
             idx_v, rows_v, accum_sh, isem, gsem, ssem):
        cid = lax.axis_index("c")
        sid = lax.axis_index("s")
        wid = cid * NS + sid
        pltpu.sync_copy(idx_hbm.at[wid, 0], idx_v.at[0])
        pltpu.sync_copy(zeros_hbm.at[pl.ds(sid * rpt, rpt)],
                        accum_sh.at[pl.ds(sid * rpt, rpt)])
        plsc.subcore_barrier()

        def outer_pair(t2, carry):
            for p in range(2):
                t = 2 * t2 + p
                nxt = pltpu.async_copy(idx_hbm.at[wid, t + 1],
                                       idx_v.at[1 - p], isem)
                gd = [
                    pltpu.async_copy(tfeats_hbm.at[idx_v.at[p, 0, b]],
                                     rows_v.at[b], gsem)
                    for b in range(inner)
                ]
                sd = []
                for b in range(inner):
                    gd[b].wait()
                    sd.append(pltpu.async_copy(rows_v.at[b],
                                               accum_sh.at[idx_v.at[p, 1, b]],
                                               ssem, add=True))
                for d in sd:
                    d.wait()
                nxt.wait()
            return carry

        lax.fori_loop(0, nouter // 2, outer_pair, 0)
        plsc.subcore_barrier()
        pltpu.sync_copy(accum_sh.at[pl.ds(sid * rpt, rpt)],
                        out_hbm.at[cid, pl.ds(sid * rpt, rpt)])

    return pl.kernel(
        body,
        out_type=jax.ShapeDtypeStruct((NC, npad, c), jnp.float32),
        mesh=mesh,
        scratch_types=[
            pltpu.VMEM((2, 2, INNER, CHUNK), jnp.int32),
            pltpu.VMEM((INNER, CHUNK, c), jnp.float32),
            pltpu.VMEM_SHARED((npad, c), jnp.float32),
            pltpu.SemaphoreType.DMA,
            pltpu.SemaphoreType.DMA,
            pltpu.SemaphoreType.DMA,
        ],
    )(idx, tflat, zeros)


def kernel(feats, in_map, out_map, W):
    n, c_in = feats.shape
    k, e = in_map.shape
    c_out = W.shape[-1]

    in32 = in_map.astype(jnp.int32)
    out32 = out_map.astype(jnp.int32)

    bn = 2000
    tfeats = pl.pallas_call(
        _matmul_body,
        grid=(k, n // bn),
        in_specs=[
            pl.BlockSpec((bn, c_in), lambda ki, ni: (ni, 0)),
            pl.BlockSpec((1, c_in, c_out), lambda ki, ni: (ki, 0, 0)),
        ],
        out_specs=pl.BlockSpec((1, bn, c_out), lambda ki, ni: (ki, ni, 0)),
        out_shape=jax.ShapeDtypeStruct((k, n, c_out), jnp.float32),
    )(feats, W)
    tflat = tfeats.reshape(k * n, c_out)

    etot = k * e
    nchunk = -(-etot // (NW * CHUNK))
    nchunk = -(-nchunk // (2 * INNER)) * (2 * INNER)
    nouter = nchunk // INNER
    ep = NW * nchunk * CHUNK
    npad = ((n + NS + 127) // 128) * 128
    rpt = npad // NS

    offs = (jnp.arange(k, dtype=jnp.int32) * n)[:, None]
    fi = (in32 + offs).reshape(-1)
    fo = out32.reshape(-1)
    fi = jnp.concatenate([fi, jnp.zeros((ep - etot,), jnp.int32)])
    fo = jnp.concatenate([fo, jnp.full((ep - etot,), n, jnp.int32)])
    fi = fi.reshape(NW, nouter, 1, INNER, CHUNK)
    fo = fo.reshape(NW, nouter, 1, INNER, CHUNK)
    idx = jnp.concatenate([fi, fo], axis=2)
    idx = jnp.concatenate(
        [idx, jnp.zeros((NW, 1, 2, INNER, CHUNK), jnp.int32)], axis=1
    )

    zeros = jnp.zeros((npad, c_out), jnp.float32)
    partials = _sc_gather_scatter(
        idx, tflat, zeros, nouter=nouter, npad=npad, c=c_out, rpt=rpt
    )

    out = pl.pallas_call(
        _add_relu_body,
        grid=(1,),
        in_specs=[
            pl.BlockSpec((1, npad, c_out), lambda i: (0, 0, 0)),
            pl.BlockSpec((1, npad, c_out), lambda i: (1, 0, 0)),
        ],
        out_specs=pl.BlockSpec((npad, c_out), lambda i: (0, 0)),
        out_shape=jax.ShapeDtypeStruct((npad, c_out), jnp.float32),
    )(partials, partials)
    return out[:n]


# speedup vs baseline: 4.6939x; 1.0862x over previous
import functools

import jax
import jax.numpy as jnp
from jax import lax
from jax.experimental import pallas as pl
from jax.experimental.pallas import tpu as pltpu
from jax.experimental.pallas import tpu_sc as plsc

NC = 2
NS = 16
NW = NC * NS
CHUNK = 128
INNER = 2
M0 = 54
M1 = 26


def _matmul_body(f_ref, w_ref, o_ref):
    o_ref[0] = jnp.dot(f_ref[...], w_ref[0], preferred_element_type=jnp.float32)


def _add_relu_body(a_ref, b_ref, o_ref):
    o_ref[...] = jnp.maximum(a_ref[0] + b_ref[0], 0.0)


@functools.partial(jax.jit, static_argnames=("npad", "c", "rpt"))
def _sc_gather_scatter(idx, tflat, zeros, *, npad, c, rpt):
    mesh = plsc.VectorSubcoreMesh(
        core_axis_name="c", subcore_axis_name="s", num_cores=NC, num_subcores=NS
    )
    inner = INNER

    def body(idx_hbm, tfeats_hbm, zeros_hbm, out_hbm,
             idx_v, rows_v, accum_sh, isem, gsem, ssem):
        cid = lax.axis_index("c")
        sid = lax.axis_index("s")
        wid = cid * NS + sid
        pltpu.sync_copy(idx_hbm.at[wid, 0], idx_v.at[0])
        pltpu.sync_copy(zeros_hbm.at[pl.ds(sid * rpt, rpt)],
                        accum_sh.at[pl.ds(sid * rpt, rpt)])
        plsc.subcore_barrier()

        def outer_pair(t2, carry):
            for p in range(2):
                t = 2 * t2 + p
                nxt = pltpu.async_copy(idx_hbm.at[wid, t + 1],
                                       idx_v.at[1 - p], isem)
                gd = [
                    pltpu.async_copy(tfeats_hbm.at[idx_v.at[p, 0, b]],
                                     rows_v.at[b], gsem)
                    for b in range(inner)
                ]
                sd = []
                for b in range(inner):
                    gd[b].wait()
                    sd.append(pltpu.async_copy(rows_v.at[b],
                                               accum_sh.at[idx_v.at[p, 1, b]],
                                               ssem, add=True))
                for d in sd:
                    d.wait()
                nxt.wait()
            return carry

        lax.fori_loop(0, lax.select(cid == 0, M0 // 2, M1 // 2), outer_pair, 0)
        plsc.subcore_barrier()
        pltpu.sync_copy(accum_sh.at[pl.ds(sid * rpt, rpt)],
                        out_hbm.at[cid, pl.ds(sid * rpt, rpt)])

    return pl.kernel(
        body,
        out_type=jax.ShapeDtypeStruct((NC, npad, c), jnp.float32),
        mesh=mesh,
        scratch_types=[
            pltpu.VMEM((2, 2, INNER, CHUNK), jnp.int32),
            pltpu.VMEM((INNER, CHUNK, c), jnp.float32),
            pltpu.VMEM_SHARED((npad, c), jnp.float32),
            pltpu.SemaphoreType.DMA,
            pltpu.SemaphoreType.DMA,
            pltpu.SemaphoreType.DMA,
        ],
    )(idx, tflat, zeros)


def kernel(feats, in_map, out_map, W):
    n, c_in = feats.shape
    k, e = in_map.shape
    c_out = W.shape[-1]

    in32 = in_map.astype(jnp.int32)
    out32 = out_map.astype(jnp.int32)

    bn = 2000
    tfeats = pl.pallas_call(
        _matmul_body,
        grid=(n // bn, k),
        in_specs=[
            pl.BlockSpec((bn, c_in), lambda ni, ki: (ni, 0)),
            pl.BlockSpec((1, c_in, c_out), lambda ni, ki: (ki, 0, 0)),
        ],
        out_specs=pl.BlockSpec((1, bn, c_out), lambda ni, ki: (ki, ni, 0)),
        out_shape=jax.ShapeDtypeStruct((k, n, c_out), jnp.float32),
    )(feats, W)
    tflat = tfeats.reshape(k * n, c_out)

    etot = k * e
    blk = INNER * CHUNK
    ep0 = NS * M0 * blk
    ep1 = NS * M1 * blk
    ep = ep0 + ep1
    assert ep >= etot
    npad = ((n + NS + 127) // 128) * 128
    rpt = npad // NS

    offs = (jnp.arange(k, dtype=jnp.int32) * n)[:, None]
    fi = (in32 + offs).reshape(-1)
    fo = out32.reshape(-1)
    fi = jnp.concatenate([fi, jnp.zeros((ep - etot,), jnp.int32)])
    fo = jnp.concatenate([fo, jnp.full((ep - etot,), n, jnp.int32)])

    def _pack(x):
        p0 = x[:ep0].reshape(NS, M0, 1, INNER, CHUNK)
        p1 = x[ep0:].reshape(NS, M1, 1, INNER, CHUNK)
        pad0 = jnp.zeros((NS, 1, 1, INNER, CHUNK), jnp.int32)
        pad1 = jnp.zeros((NS, M0 - M1 + 1, 1, INNER, CHUNK), jnp.int32)
        return jnp.concatenate(
            [jnp.concatenate([p0, pad0], axis=1),
             jnp.concatenate([p1, pad1], axis=1)], axis=0)

    idx = jnp.concatenate([_pack(fi), _pack(fo)], axis=2)

    zeros = jnp.zeros((npad, c_out), jnp.float32)
    partials = _sc_gather_scatter(
        idx, tflat, zeros, npad=npad, c=c_out, rpt=rpt
    )

    out = pl.pallas_call(
        _add_relu_body,
        grid=(1,),
        in_specs=[
            pl.BlockSpec((1, npad, c_out), lambda i: (0, 0, 0)),
            pl.BlockSpec((1, npad, c_out), lambda i: (1, 0, 0)),
        ],
        out_specs=pl.BlockSpec((npad, c_out), lambda i: (0, 0)),
        out_shape=jax.ShapeDtypeStruct((npad, c_out), jnp.float32),
    )(partials, partials)
    return out[:n]
